# Initial kernel scaffold; baseline (speedup 1.0000x reference)
import functools

import jax
import jax.numpy as jnp
from jax.experimental import pallas as pl
from jax.experimental.pallas import tpu as pltpu

EPS = 1e-5
LOG_EPS = -11.512925464970229
OUT_LO, OUT_HI = 0.0, 1.0
NPTS = 100
TOP_N = 8
NEG = -1.0e30
IBIG = jnp.int32(2**30)


def _body(x_ref, cs_ref, ss_ref, oc_ref, os_ref, ant_ref, cons_ref, out_ref,
          *, bt, d, m, r, m_out):
    mm = m + 1
    k = d * mm

    x = x_ref[...]
    pieces = []
    for i in range(d):
        xi = x[:, i:i + 1]
        ci = cs_ref[i:i + 1, :]
        si = ss_ref[i:i + 1, :]
        t = -((xi - ci) ** 2) / (2.0 * si * si)
        t = jnp.clip(t, LOG_EPS, 0.0)
        pieces.append(t)
        pieces.append(jnp.zeros((bt, 1), jnp.float32))
    logF = jnp.concatenate(pieces, axis=1)

    opieces = []
    for i in range(d):
        ai = ant_ref[i:i + 1, :]
        ai = jnp.where(ai < 0, m, ai)
        js = jax.lax.broadcasted_iota(jnp.int32, (mm, r), 0)
        opieces.append((js == ai).astype(jnp.float32))
    onehot = jnp.concatenate(opieces, axis=0)

    logf = jax.lax.dot_general(
        logF, onehot, (((1,), (0,)), ((), ())),
        preferred_element_type=jnp.float32)

    cons = cons_ref[...]
    ridx = jax.lax.broadcasted_iota(jnp.int32, (1, r), 1)
    packed = ridx * 8 + cons

    vals = logf
    tops_v, tops_p = [], []
    for _ in range(TOP_N):
        mx = jnp.max(vals, axis=1, keepdims=True)
        cand = jnp.where(vals == mx, packed, IBIG)
        sel = jnp.min(cand, axis=1, keepdims=True)
        tops_v.append(mx)
        tops_p.append(sel)
        vals = jnp.where(cand == sel, NEG, vals)
    top_v = jnp.concatenate(tops_v, axis=1)
    top_p = jnp.concatenate(tops_p, axis=1)
    cons_sel = jnp.bitwise_and(top_p, 7)
    firing = jnp.exp(top_v)

    pidx = jax.lax.broadcasted_iota(jnp.int32, (1, 128), 1)
    u = pidx.astype(jnp.float32) * ((OUT_HI - OUT_LO) / (NPTS - 1)) + OUT_LO
    valid = pidx < NPTS
    s1_t = jnp.zeros((bt, TOP_N), jnp.float32)
    s0_t = jnp.zeros((bt, TOP_N), jnp.float32)
    for j in range(m_out):
        cj = oc_ref[0:1, j:j + 1]
        sj = os_ref[0:1, j:j + 1]
        e = jnp.exp(-((u - cj) ** 2) / (2.0 * sj * sj))
        e = jnp.where(valid, e, 0.0)
        s0 = jnp.sum(e, axis=1, keepdims=True)
        s1 = jnp.sum(u * e, axis=1, keepdims=True)
        hit = cons_sel == j
        s1_t = jnp.where(hit, s1, s1_t)
        s0_t = jnp.where(hit, s0, s0_t)

    num = jnp.sum(firing * s1_t, axis=1, keepdims=True)
    den = jnp.sum(firing * s0_t, axis=1, keepdims=True) + EPS
    out_ref[...] = num / den


def kernel(x, centers, sigmas, out_centers, out_sigmas, antecedents,
           consequents):
    b, d = x.shape
    m = centers.shape[1]
    r = antecedents.shape[0]
    m_out = out_centers.shape[0]
    bt = 128

    ant_t = antecedents.T
    cons2d = consequents.reshape(1, r)
    oc2 = out_centers.reshape(1, m_out)
    os2 = out_sigmas.reshape(1, m_out)

    body = functools.partial(_body, bt=bt, d=d, m=m, r=r, m_out=m_out)
    out = pl.pallas_call(
        body,
        grid=(b // bt,),
        in_specs=[
            pl.BlockSpec((bt, d), lambda i: (i, 0)),
            pl.BlockSpec((d, m), lambda i: (0, 0)),
            pl.BlockSpec((d, m), lambda i: (0, 0)),
            pl.BlockSpec((1, m_out), lambda i: (0, 0)),
            pl.BlockSpec((1, m_out), lambda i: (0, 0)),
            pl.BlockSpec((d, r), lambda i: (0, 0)),
            pl.BlockSpec((1, r), lambda i: (0, 0)),
        ],
        out_specs=pl.BlockSpec((bt, 1), lambda i: (i, 0)),
        out_shape=jax.ShapeDtypeStruct((b, 1), jnp.float32),
        compiler_params=pltpu.CompilerParams(
            dimension_semantics=("parallel",)),
    )(x, centers, sigmas, oc2, os2, ant_t, cons2d)
    return out.reshape(b)


# speedup vs baseline: 8.0069x; 8.0069x over previous
import functools

import jax
import jax.numpy as jnp
from jax.experimental import pallas as pl
from jax.experimental.pallas import tpu as pltpu

EPS = 1e-5
LOG_EPS = -11.512925464970229
OUT_LO, OUT_HI = 0.0, 1.0
NPTS = 100
TOP_N = 8
NEG = -1.0e30
IBIG = 2**30


def _body(x_ref, cs_ref, ss_ref, oc_ref, os_ref, ant_ref, cons_ref, out_ref,
          *, bt, d, m, r, m_out):
    mm = m + 1
    k = d * mm

    x = x_ref[...]
    pieces = []
    for i in range(d):
        xi = x[:, i:i + 1]
        ci = cs_ref[i:i + 1, :]
        si = ss_ref[i:i + 1, :]
        t = -((xi - ci) ** 2) / (2.0 * si * si)
        t = jnp.clip(t, LOG_EPS, 0.0)
        pieces.append(t)
        pieces.append(jnp.zeros((bt, 1), jnp.float32))
    logF = jnp.concatenate(pieces, axis=1)

    opieces = []
    for i in range(d):
        ai = ant_ref[i:i + 1, :]
        ai = jnp.where(ai < 0, m, ai)
        js = jax.lax.broadcasted_iota(jnp.int32, (mm, r), 0)
        opieces.append((js == ai).astype(jnp.float32))
    onehot = jnp.concatenate(opieces, axis=0)

    logf = jax.lax.dot_general(
        logF, onehot, (((1,), (0,)), ((), ())),
        precision=jax.lax.Precision.HIGHEST,
        preferred_element_type=jnp.float32)

    cons = cons_ref[...]
    ridx = jax.lax.broadcasted_iota(jnp.int32, (1, r), 1)
    packed = ridx * 8 + cons

    vals = logf
    tops_v, tops_p = [], []
    for _ in range(TOP_N):
        mx = jnp.max(vals, axis=1, keepdims=True)
        cand = jnp.where(vals == mx, packed, IBIG)
        sel = jnp.min(cand, axis=1, keepdims=True)
        tops_v.append(mx)
        tops_p.append(sel)
        vals = jnp.where(cand == sel, NEG, vals)
    top_v = jnp.concatenate(tops_v, axis=1)
    top_p = jnp.concatenate(tops_p, axis=1)
    cons_sel = jnp.bitwise_and(top_p, 7)
    firing = jnp.exp(top_v)

    pidx = jax.lax.broadcasted_iota(jnp.int32, (1, 128), 1)
    u = pidx.astype(jnp.float32) * ((OUT_HI - OUT_LO) / (NPTS - 1)) + OUT_LO
    valid = pidx < NPTS
    s1_t = jnp.zeros((bt, TOP_N), jnp.float32)
    s0_t = jnp.zeros((bt, TOP_N), jnp.float32)
    for j in range(m_out):
        cj = oc_ref[0:1, j:j + 1]
        sj = os_ref[0:1, j:j + 1]
        e = jnp.exp(-((u - cj) ** 2) / (2.0 * sj * sj))
        e = jnp.where(valid, e, 0.0)
        s0 = jnp.sum(e, axis=1, keepdims=True)
        s1 = jnp.sum(u * e, axis=1, keepdims=True)
        hit = cons_sel == j
        s1_t = jnp.where(hit, s1, s1_t)
        s0_t = jnp.where(hit, s0, s0_t)

    num = jnp.sum(firing * s1_t, axis=1, keepdims=True)
    den = jnp.sum(firing * s0_t, axis=1, keepdims=True) + EPS
    out_ref[...] = num / den


def kernel(x, centers, sigmas, out_centers, out_sigmas, antecedents,
           consequents):
    b, d = x.shape
    m = centers.shape[1]
    r = antecedents.shape[0]
    m_out = out_centers.shape[0]
    bt = 128

    ant_t = antecedents.T
    cons2d = consequents.reshape(1, r)
    oc2 = out_centers.reshape(1, m_out)
    os2 = out_sigmas.reshape(1, m_out)

    body = functools.partial(_body, bt=bt, d=d, m=m, r=r, m_out=m_out)
    out = pl.pallas_call(
        body,
        grid=(b // bt,),
        in_specs=[
            pl.BlockSpec((bt, d), lambda i: (i, 0)),
            pl.BlockSpec((d, m), lambda i: (0, 0)),
            pl.BlockSpec((d, m), lambda i: (0, 0)),
            pl.BlockSpec((1, m_out), lambda i: (0, 0)),
            pl.BlockSpec((1, m_out), lambda i: (0, 0)),
            pl.BlockSpec((d, r), lambda i: (0, 0)),
            pl.BlockSpec((1, r), lambda i: (0, 0)),
        ],
        out_specs=pl.BlockSpec((bt, 1), lambda i: (i, 0)),
        out_shape=jax.ShapeDtypeStruct((b, 1), jnp.float32),
        compiler_params=pltpu.CompilerParams(
            dimension_semantics=("parallel",)),
    )(x, centers, sigmas, oc2, os2, ant_t, cons2d)
    return out.reshape(b)
